# Initial kernel scaffold; baseline (speedup 1.0000x reference)
import jax
import jax.numpy as jnp
from jax.experimental import pallas as pl


def kernel(x, edge_index, edge_weight_params, W1, b1, W2, b2, W3, b3, Wlin, blin):
    raise NotImplementedError("write your pallas kernel here")


# speedup vs baseline: 7.2027x; 7.2027x over previous
import functools

import jax
import jax.numpy as jnp
from jax import lax
from jax.experimental import pallas as pl
from jax.experimental.pallas import tpu as pltpu
from jax.experimental.pallas import tpu_sc as plsc

N = 10000
E = 320000
F = 128
NCLS = 16
NC = 2
NS = 16
NW = NC * NS
K = 80
FH = F // NC
E_PAD = 327680
PAD_E = E_PAD - E
EB = E_PAD // K
BPT = EB // NW
BPC = EB // NS
N_ACC = 10240
NPT = N_ACC // NS
NZC = 128
N_PAD = N_ACC
NPS = N_PAD // NS

_mesh = plsc.VectorSubcoreMesh(core_axis_name="c", subcore_axis_name="s")


@functools.partial(
    pl.kernel,
    out_type=(
        jax.ShapeDtypeStruct((EB, K), jnp.float32),
        jax.ShapeDtypeStruct((NC * N_PAD,), jnp.float32),
        jax.ShapeDtypeStruct((NC * N_PAD,), jnp.float32),
    ),
    mesh=_mesh,
    compiler_params=pltpu.CompilerParams(use_tc_tiling_on_sc=False),
    scratch_types=(
        pltpu.VMEM((BPT, K), jnp.float32),
        pltpu.VMEM((BPT, K), jnp.int32),
        pltpu.VMEM((BPT, K), jnp.float32),
        pltpu.VMEM((K,), jnp.float32),
        pltpu.VMEM((NPS,), jnp.float32),
        pltpu.VMEM_SHARED((N_PAD,), jnp.float32),
        pltpu.VMEM_SHARED((N_PAD,), jnp.float32),
    ),
)
def _deg_kernel(params2_h, dst2_h, ew2_h, deg1_h, deg3_h,
                pbuf, dbuf, ebuf, ones_v, zb, acc1, acc3):
    c = lax.axis_index("c")
    s = lax.axis_index("s")
    wid = s * NC + c

    def zbody(i, carry):
        zb[pl.ds(i * 16, 16)] = jnp.zeros((16,), jnp.float32)
        return carry

    lax.fori_loop(0, NPS // 16, zbody, 0)
    for i in range(K // 16):
        ones_v[pl.ds(i * 16, 16)] = jnp.full((16,), 1.0, jnp.float32)

    pltpu.sync_copy(zb, acc1.at[pl.ds(s * NPS, NPS)])
    pltpu.sync_copy(zb, acc3.at[pl.ds(s * NPS, NPS)])
    plsc.subcore_barrier()

    row0 = wid * BPT
    pltpu.sync_copy(params2_h.at[pl.ds(row0, BPT)], pbuf)
    pltpu.sync_copy(dst2_h.at[pl.ds(row0, BPT)], dbuf)

    def body(b, carry):
        for cc in range(K // 16):
            sl = pl.ds(cc * 16, 16)
            v = pbuf[b, sl]
            ebuf[b, sl] = 1.0 / (1.0 + jnp.exp(-v))
        pltpu.sync_copy(ebuf.at[b], acc1.at[dbuf.at[b]], add=True)
        pltpu.sync_copy(ones_v, acc3.at[dbuf.at[b]], add=True)
        return carry

    lax.fori_loop(0, BPT, body, 0)
    pltpu.sync_copy(ebuf, ew2_h.at[pl.ds(row0, BPT)])
    plsc.subcore_barrier()

    base = c * N_PAD + s * NPS
    pltpu.sync_copy(acc1.at[pl.ds(s * NPS, NPS)], zb)
    pltpu.sync_copy(zb, deg1_h.at[pl.ds(base, NPS)])
    pltpu.sync_copy(acc3.at[pl.ds(s * NPS, NPS)], zb)
    pltpu.sync_copy(zb, deg3_h.at[pl.ds(base, NPS)])


def _make_spmm(weighted):
    @functools.partial(
        pl.kernel,
        out_type=jax.ShapeDtypeStruct((NC, N_ACC, FH), jnp.float32),
        mesh=_mesh,
        compiler_params=pltpu.CompilerParams(use_tc_tiling_on_sc=False),
        scratch_types=(
            pltpu.VMEM((BPC, K), jnp.int32),
            pltpu.VMEM((BPC, K), jnp.int32),
            pltpu.VMEM((BPC, K), jnp.float32),
            pltpu.VMEM((K, FH), jnp.float32),
            pltpu.VMEM((NZC, FH), jnp.float32),
            pltpu.VMEM_SHARED((N_ACC, FH), jnp.float32),
            pltpu.SemaphoreType.DMA,
        ),
    )
    def spmm(y_h, src2_h, dst2_h, w2_h, s_h, sbuf, dbuf, wbuf, rows, zb, acc, sem):
        c = lax.axis_index("c")
        s_idx = lax.axis_index("s")

        def zrow(i, carry):
            for cc in range(FH // 16):
                zb[i, pl.ds(cc * 16, 16)] = jnp.zeros((16,), jnp.float32)
            return carry

        lax.fori_loop(0, NZC, zrow, 0)
        r0 = s_idx * NPT
        for j in range(NPT // NZC):
            pltpu.sync_copy(zb, acc.at[pl.ds(r0 + j * NZC, NZC)])
        plsc.subcore_barrier()

        row0 = s_idx * BPC
        pltpu.sync_copy(src2_h.at[pl.ds(row0, BPC)], sbuf)
        pltpu.sync_copy(dst2_h.at[pl.ds(row0, BPC)], dbuf)
        if weighted:
            pltpu.sync_copy(w2_h.at[pl.ds(row0, BPC)], wbuf)

        def body(b, carry):
            pltpu.async_copy(y_h.at[c].at[sbuf.at[b]], rows, sem).wait()
            if weighted:
                def scale(g, inner):
                    wv = wbuf[b, pl.ds(g * 16, 16)]
                    for j in range(16):
                        wj = wv[j]
                        for cc in range(FH // 16):
                            sl = pl.ds(cc * 16, 16)
                            rows[g * 16 + j, sl] = rows[g * 16 + j, sl] * wj
                    return inner

                lax.fori_loop(0, K // 16, scale, 0)
            pltpu.sync_copy(rows, acc.at[dbuf.at[b]], add=True)
            return carry

        lax.fori_loop(0, BPC, body, 0)
        plsc.subcore_barrier()

        for j in range(NPT // NZC):
            sl = pl.ds(r0 + j * NZC, NZC)
            pltpu.sync_copy(acc.at[sl], zb)
            pltpu.sync_copy(zb, s_h.at[c, sl])

    return spmm


_spmm_w = _make_spmm(True)
_spmm_u = _make_spmm(False)


BR = 1000


def _tc_a_body(deg1_r, deg3_r, x_r, w1_r, d1_o, d3_o, y0_o):
    d1 = lax.rsqrt(deg1_r[:, 0:1] + deg1_r[:, 1:2] + 1.0)
    d3 = lax.rsqrt(deg3_r[:, 0:1] + deg3_r[:, 1:2] + 1.0)
    d1_o[...] = d1
    d3_o[...] = d3
    y0 = d1 * jnp.dot(x_r[...], w1_r[...], preferred_element_type=jnp.float32)
    y0_o[0] = y0[:, :FH]
    y0_o[1] = y0[:, FH:]


def _tc_a(deg1T, deg3T, x, W1):
    return pl.pallas_call(
        _tc_a_body,
        grid=(N // BR,),
        in_specs=[
            pl.BlockSpec((BR, NC), lambda i: (i, 0)),
            pl.BlockSpec((BR, NC), lambda i: (i, 0)),
            pl.BlockSpec((BR, F), lambda i: (i, 0)),
            pl.BlockSpec((F, F), lambda i: (0, 0)),
        ],
        out_specs=[
            pl.BlockSpec((BR, 1), lambda i: (i, 0)),
            pl.BlockSpec((BR, 1), lambda i: (i, 0)),
            pl.BlockSpec((NC, BR, FH), lambda i: (0, i, 0)),
        ],
        out_shape=[
            jax.ShapeDtypeStruct((N, 1), jnp.float32),
            jax.ShapeDtypeStruct((N, 1), jnp.float32),
            jax.ShapeDtypeStruct((NC, N, FH), jnp.float32),
        ],
    )(deg1T, deg3T, x, W1)


def _tc_b_body(s_r, y_r, dp_r, b_r, w_r, dn_r, x_o, yn_o):
    comb = jnp.concatenate([s_r[0] + y_r[0], s_r[1] + y_r[1]], axis=1)
    xl = jnp.maximum(dp_r[...] * comb + b_r[...], 0.0)
    x_o[...] = xl
    yn = dn_r[...] * jnp.dot(xl, w_r[...], preferred_element_type=jnp.float32)
    yn_o[0] = yn[:, :FH]
    yn_o[1] = yn[:, FH:]


def _tc_b(s, y_prev, d_prev, b_prev, W_next, d_next):
    return pl.pallas_call(
        _tc_b_body,
        grid=(N // BR,),
        in_specs=[
            pl.BlockSpec((NC, BR, FH), lambda i: (0, i, 0)),
            pl.BlockSpec((NC, BR, FH), lambda i: (0, i, 0)),
            pl.BlockSpec((BR, 1), lambda i: (i, 0)),
            pl.BlockSpec((1, F), lambda i: (0, 0)),
            pl.BlockSpec((F, F), lambda i: (0, 0)),
            pl.BlockSpec((BR, 1), lambda i: (i, 0)),
        ],
        out_specs=[
            pl.BlockSpec((BR, F), lambda i: (i, 0)),
            pl.BlockSpec((NC, BR, FH), lambda i: (0, i, 0)),
        ],
        out_shape=[
            jax.ShapeDtypeStruct((N, F), jnp.float32),
            jax.ShapeDtypeStruct((NC, N, FH), jnp.float32),
        ],
    )(s, y_prev, d_prev, b_prev, W_next, d_next)


def _tc_d_body(s_r, y_r, d3_r, b_r, x1_r, x2_r, wl_r, bl_r, out_o):
    x3 = d3_r[...] * jnp.concatenate(
        [s_r[0] + y_r[0], s_r[1] + y_r[1]], axis=1) + b_r[...]
    wl = wl_r[...]
    h = (jnp.dot(x1_r[...], wl[0:F], preferred_element_type=jnp.float32)
         + jnp.dot(x2_r[...], wl[F:2 * F], preferred_element_type=jnp.float32)
         + jnp.dot(x3, wl[2 * F:3 * F], preferred_element_type=jnp.float32)
         + bl_r[...])
    m = jnp.max(h, axis=1, keepdims=True)
    e = jnp.exp(h - m)
    lse = jnp.log(jnp.sum(e, axis=1, keepdims=True))
    out_o[...] = h - m - lse


def _tc_d(s3, y2, d3, b3, x1, x2, Wlin, blin):
    return pl.pallas_call(
        _tc_d_body,
        grid=(N // BR,),
        in_specs=[
            pl.BlockSpec((NC, BR, FH), lambda i: (0, i, 0)),
            pl.BlockSpec((NC, BR, FH), lambda i: (0, i, 0)),
            pl.BlockSpec((BR, 1), lambda i: (i, 0)),
            pl.BlockSpec((1, F), lambda i: (0, 0)),
            pl.BlockSpec((BR, F), lambda i: (i, 0)),
            pl.BlockSpec((BR, F), lambda i: (i, 0)),
            pl.BlockSpec((3 * F, NCLS), lambda i: (0, 0)),
            pl.BlockSpec((1, NCLS), lambda i: (0, 0)),
        ],
        out_specs=pl.BlockSpec((BR, NCLS), lambda i: (i, 0)),
        out_shape=jax.ShapeDtypeStruct((N, NCLS), jnp.float32),
    )(s3, y2, d3, b3, x1, x2, Wlin, blin)


def kernel(x, edge_index, edge_weight_params, W1, b1, W2, b2, W3, b3, Wlin, blin):
    pad_src = jnp.zeros((PAD_E,), jnp.int32)
    pad_dst = jnp.full((PAD_E,), N, jnp.int32)
    src2 = jnp.concatenate([edge_index[0], pad_src]).reshape(EB, K)
    dst2 = jnp.concatenate([edge_index[1], pad_dst]).reshape(EB, K)
    p2 = jnp.concatenate(
        [edge_weight_params, jnp.zeros((PAD_E,), jnp.float32)]).reshape(EB, K)

    ew2, deg1f, deg3f = _deg_kernel(p2, dst2)
    deg1T = deg1f.reshape(NC, N_PAD)[:, :N].T
    deg3T = deg3f.reshape(NC, N_PAD)[:, :N].T

    d1, d3, y0 = _tc_a(deg1T, deg3T, x, W1)

    s1 = _spmm_w(y0, src2, dst2, ew2)
    x1, y1 = _tc_b(s1, y0, d1, b1.reshape(1, F), W2, d1)

    s2 = _spmm_w(y1, src2, dst2, ew2)
    x2, y2 = _tc_b(s2, y1, d1, b2.reshape(1, F), W3, d3)

    s3 = _spmm_u(y2, src2, dst2, ew2)
    out = _tc_d(s3, y2, d3, b3.reshape(1, F), x1, x2,
                Wlin, blin.reshape(1, NCLS))
    return out


# speedup vs baseline: 9.8582x; 1.3687x over previous
import functools

import jax
import jax.numpy as jnp
from jax import lax
from jax.experimental import pallas as pl
from jax.experimental.pallas import tpu as pltpu
from jax.experimental.pallas import tpu_sc as plsc

N = 10000
E = 320000
F = 128
NCLS = 16
NC = 2
NS = 16
NW = NC * NS
K = 80
FH = F // NC
E_PAD = 327680
PAD_E = E_PAD - E
EB = E_PAD // K
BPT = EB // NW
BPC = EB // NS
N_ACC = 10240
NPT = N_ACC // NS
NZC = 128
N_PAD = N_ACC
NPS = N_PAD // NS

_mesh = plsc.VectorSubcoreMesh(core_axis_name="c", subcore_axis_name="s")


@functools.partial(
    pl.kernel,
    out_type=(
        jax.ShapeDtypeStruct((EB, K), jnp.float32),
        jax.ShapeDtypeStruct((NC * N_PAD,), jnp.float32),
        jax.ShapeDtypeStruct((NC * N_PAD,), jnp.float32),
    ),
    mesh=_mesh,
    compiler_params=pltpu.CompilerParams(use_tc_tiling_on_sc=False),
    scratch_types=(
        pltpu.VMEM((BPT, K), jnp.float32),
        pltpu.VMEM((BPT, K), jnp.int32),
        pltpu.VMEM((BPT, K), jnp.float32),
        pltpu.VMEM((K,), jnp.float32),
        pltpu.VMEM((NPS,), jnp.float32),
        pltpu.VMEM_SHARED((N_PAD,), jnp.float32),
        pltpu.VMEM_SHARED((N_PAD,), jnp.float32),
    ),
)
def _deg_kernel(params2_h, dst2_h, ew2_h, deg1_h, deg3_h,
                pbuf, dbuf, ebuf, ones_v, zb, acc1, acc3):
    c = lax.axis_index("c")
    s = lax.axis_index("s")
    wid = s * NC + c

    def zbody(i, carry):
        zb[pl.ds(i * 16, 16)] = jnp.zeros((16,), jnp.float32)
        return carry

    lax.fori_loop(0, NPS // 16, zbody, 0)
    for i in range(K // 16):
        ones_v[pl.ds(i * 16, 16)] = jnp.full((16,), 1.0, jnp.float32)

    pltpu.sync_copy(zb, acc1.at[pl.ds(s * NPS, NPS)])
    pltpu.sync_copy(zb, acc3.at[pl.ds(s * NPS, NPS)])
    plsc.subcore_barrier()

    row0 = wid * BPT
    pltpu.sync_copy(params2_h.at[pl.ds(row0, BPT)], pbuf)
    pltpu.sync_copy(dst2_h.at[pl.ds(row0, BPT)], dbuf)

    def body(b, carry):
        for cc in range(K // 16):
            sl = pl.ds(cc * 16, 16)
            v = pbuf[b, sl]
            ebuf[b, sl] = 1.0 / (1.0 + jnp.exp(-v))
        pltpu.sync_copy(ebuf.at[b], acc1.at[dbuf.at[b]], add=True)
        pltpu.sync_copy(ones_v, acc3.at[dbuf.at[b]], add=True)
        return carry

    lax.fori_loop(0, BPT, body, 0)
    pltpu.sync_copy(ebuf, ew2_h.at[pl.ds(row0, BPT)])
    plsc.subcore_barrier()

    base = c * N_PAD + s * NPS
    pltpu.sync_copy(acc1.at[pl.ds(s * NPS, NPS)], zb)
    pltpu.sync_copy(zb, deg1_h.at[pl.ds(base, NPS)])
    pltpu.sync_copy(acc3.at[pl.ds(s * NPS, NPS)], zb)
    pltpu.sync_copy(zb, deg3_h.at[pl.ds(base, NPS)])


def _make_spmm(weighted):
    @functools.partial(
        pl.kernel,
        out_type=jax.ShapeDtypeStruct((NC, N_ACC, FH), jnp.float32),
        mesh=_mesh,
        compiler_params=pltpu.CompilerParams(use_tc_tiling_on_sc=False),
        scratch_types=(
            pltpu.VMEM((BPC, K), jnp.int32),
            pltpu.VMEM((BPC, K), jnp.int32),
            pltpu.VMEM((BPC, K), jnp.float32),
            pltpu.VMEM((K, FH), jnp.float32),
            pltpu.VMEM((K, FH), jnp.float32),
            pltpu.VMEM((NZC, FH), jnp.float32),
            pltpu.VMEM_SHARED((N_ACC, FH), jnp.float32),
            pltpu.SemaphoreType.DMA,
            pltpu.SemaphoreType.DMA,
        ),
    )
    def spmm(y_h, src2_h, dst2_h, w2_h, s_h,
             sbuf, dbuf, wbuf, rows0, rows1, zb, acc, sem0, sem1):
        c = lax.axis_index("c")
        s_idx = lax.axis_index("s")

        def zrow(i, carry):
            for cc in range(FH // 16):
                zb[i, pl.ds(cc * 16, 16)] = jnp.zeros((16,), jnp.float32)
            return carry

        lax.fori_loop(0, NZC, zrow, 0)
        r0 = s_idx * NPT
        for j in range(NPT // NZC):
            pltpu.sync_copy(zb, acc.at[pl.ds(r0 + j * NZC, NZC)])
        plsc.subcore_barrier()

        row0 = s_idx * BPC
        pltpu.sync_copy(src2_h.at[pl.ds(row0, BPC)], sbuf)
        pltpu.sync_copy(dst2_h.at[pl.ds(row0, BPC)], dbuf)
        if weighted:
            pltpu.sync_copy(w2_h.at[pl.ds(row0, BPC)], wbuf)

        def scale_rows(rows, b):
            def scale(g, inner):
                wv = wbuf[b, pl.ds(g * 16, 16)]
                for j in range(16):
                    wj = wv[j]
                    for cc in range(FH // 16):
                        sl = pl.ds(cc * 16, 16)
                        rows[g * 16 + j, sl] = rows[g * 16 + j, sl] * wj
                return inner

            lax.fori_loop(0, K // 16, scale, 0)

        def stage(b, rows, sem, rows_nxt, sem_nxt):
            pltpu.make_async_copy(y_h.at[c].at[sbuf.at[b]], rows, sem).wait()

            @pl.when(b + 1 < BPC)
            def _():
                pltpu.async_copy(y_h.at[c].at[sbuf.at[b + 1]], rows_nxt,
                                 sem_nxt)

            if weighted:
                scale_rows(rows, b)
            pltpu.sync_copy(rows, acc.at[dbuf.at[b]], add=True)

        pltpu.async_copy(y_h.at[c].at[sbuf.at[0]], rows0, sem0)

        def body(i2, carry):
            b0 = i2 * 2
            stage(b0, rows0, sem0, rows1, sem1)
            stage(b0 + 1, rows1, sem1, rows0, sem0)
            return carry

        lax.fori_loop(0, BPC // 2, body, 0)
        plsc.subcore_barrier()

        for j in range(NPT // NZC):
            sl = pl.ds(r0 + j * NZC, NZC)
            pltpu.sync_copy(acc.at[sl], zb)
            pltpu.sync_copy(zb, s_h.at[c, sl])

    return spmm


_spmm_w = _make_spmm(True)
_spmm_u = _make_spmm(False)


BR = 1000


def _tc_a_body(deg1_r, deg3_r, x_r, w1_r, d1_o, d3_o, y0_o):
    d1 = lax.rsqrt(deg1_r[:, 0:1] + deg1_r[:, 1:2] + 1.0)
    d3 = lax.rsqrt(deg3_r[:, 0:1] + deg3_r[:, 1:2] + 1.0)
    d1_o[...] = d1
    d3_o[...] = d3
    y0 = d1 * jnp.dot(x_r[...], w1_r[...], preferred_element_type=jnp.float32)
    y0_o[0] = y0[:, :FH]
    y0_o[1] = y0[:, FH:]


def _tc_a(deg1T, deg3T, x, W1):
    return pl.pallas_call(
        _tc_a_body,
        grid=(N // BR,),
        in_specs=[
            pl.BlockSpec((BR, NC), lambda i: (i, 0)),
            pl.BlockSpec((BR, NC), lambda i: (i, 0)),
            pl.BlockSpec((BR, F), lambda i: (i, 0)),
            pl.BlockSpec((F, F), lambda i: (0, 0)),
        ],
        out_specs=[
            pl.BlockSpec((BR, 1), lambda i: (i, 0)),
            pl.BlockSpec((BR, 1), lambda i: (i, 0)),
            pl.BlockSpec((NC, BR, FH), lambda i: (0, i, 0)),
        ],
        out_shape=[
            jax.ShapeDtypeStruct((N, 1), jnp.float32),
            jax.ShapeDtypeStruct((N, 1), jnp.float32),
            jax.ShapeDtypeStruct((NC, N, FH), jnp.float32),
        ],
    )(deg1T, deg3T, x, W1)


def _tc_b_body(s_r, y_r, dp_r, b_r, w_r, dn_r, x_o, yn_o):
    comb = jnp.concatenate([s_r[0] + y_r[0], s_r[1] + y_r[1]], axis=1)
    xl = jnp.maximum(dp_r[...] * comb + b_r[...], 0.0)
    x_o[...] = xl
    yn = dn_r[...] * jnp.dot(xl, w_r[...], preferred_element_type=jnp.float32)
    yn_o[0] = yn[:, :FH]
    yn_o[1] = yn[:, FH:]


def _tc_b(s, y_prev, d_prev, b_prev, W_next, d_next):
    return pl.pallas_call(
        _tc_b_body,
        grid=(N // BR,),
        in_specs=[
            pl.BlockSpec((NC, BR, FH), lambda i: (0, i, 0)),
            pl.BlockSpec((NC, BR, FH), lambda i: (0, i, 0)),
            pl.BlockSpec((BR, 1), lambda i: (i, 0)),
            pl.BlockSpec((1, F), lambda i: (0, 0)),
            pl.BlockSpec((F, F), lambda i: (0, 0)),
            pl.BlockSpec((BR, 1), lambda i: (i, 0)),
        ],
        out_specs=[
            pl.BlockSpec((BR, F), lambda i: (i, 0)),
            pl.BlockSpec((NC, BR, FH), lambda i: (0, i, 0)),
        ],
        out_shape=[
            jax.ShapeDtypeStruct((N, F), jnp.float32),
            jax.ShapeDtypeStruct((NC, N, FH), jnp.float32),
        ],
    )(s, y_prev, d_prev, b_prev, W_next, d_next)


def _tc_d_body(s_r, y_r, d3_r, b_r, x1_r, x2_r, wl_r, bl_r, out_o):
    x3 = d3_r[...] * jnp.concatenate(
        [s_r[0] + y_r[0], s_r[1] + y_r[1]], axis=1) + b_r[...]
    wl = wl_r[...]
    h = (jnp.dot(x1_r[...], wl[0:F], preferred_element_type=jnp.float32)
         + jnp.dot(x2_r[...], wl[F:2 * F], preferred_element_type=jnp.float32)
         + jnp.dot(x3, wl[2 * F:3 * F], preferred_element_type=jnp.float32)
         + bl_r[...])
    m = jnp.max(h, axis=1, keepdims=True)
    e = jnp.exp(h - m)
    lse = jnp.log(jnp.sum(e, axis=1, keepdims=True))
    out_o[...] = h - m - lse


def _tc_d(s3, y2, d3, b3, x1, x2, Wlin, blin):
    return pl.pallas_call(
        _tc_d_body,
        grid=(N // BR,),
        in_specs=[
            pl.BlockSpec((NC, BR, FH), lambda i: (0, i, 0)),
            pl.BlockSpec((NC, BR, FH), lambda i: (0, i, 0)),
            pl.BlockSpec((BR, 1), lambda i: (i, 0)),
            pl.BlockSpec((1, F), lambda i: (0, 0)),
            pl.BlockSpec((BR, F), lambda i: (i, 0)),
            pl.BlockSpec((BR, F), lambda i: (i, 0)),
            pl.BlockSpec((3 * F, NCLS), lambda i: (0, 0)),
            pl.BlockSpec((1, NCLS), lambda i: (0, 0)),
        ],
        out_specs=pl.BlockSpec((BR, NCLS), lambda i: (i, 0)),
        out_shape=jax.ShapeDtypeStruct((N, NCLS), jnp.float32),
    )(s3, y2, d3, b3, x1, x2, Wlin, blin)


def kernel(x, edge_index, edge_weight_params, W1, b1, W2, b2, W3, b3, Wlin, blin):
    pad_src = jnp.zeros((PAD_E,), jnp.int32)
    pad_dst = jnp.full((PAD_E,), N, jnp.int32)
    src2 = jnp.concatenate([edge_index[0], pad_src]).reshape(EB, K)
    dst2 = jnp.concatenate([edge_index[1], pad_dst]).reshape(EB, K)
    p2 = jnp.concatenate(
        [edge_weight_params, jnp.zeros((PAD_E,), jnp.float32)]).reshape(EB, K)

    ew2, deg1f, deg3f = _deg_kernel(p2, dst2)
    deg1T = deg1f.reshape(NC, N_PAD)[:, :N].T
    deg3T = deg3f.reshape(NC, N_PAD)[:, :N].T

    d1, d3, y0 = _tc_a(deg1T, deg3T, x, W1)

    s1 = _spmm_w(y0, src2, dst2, ew2)
    x1, y1 = _tc_b(s1, y0, d1, b1.reshape(1, F), W2, d1)

    s2 = _spmm_w(y1, src2, dst2, ew2)
    x2, y2 = _tc_b(s2, y1, d1, b2.reshape(1, F), W3, d3)

    s3 = _spmm_u(y2, src2, dst2, ew2)
    out = _tc_d(s3, y2, d3, b3.reshape(1, F), x1, x2,
                Wlin, blin.reshape(1, NCLS))
    return out


# speedup vs baseline: 10.4399x; 1.0590x over previous
import functools

import jax
import jax.numpy as jnp
from jax import lax
from jax.experimental import pallas as pl
from jax.experimental.pallas import tpu as pltpu
from jax.experimental.pallas import tpu_sc as plsc

N = 10000
E = 320000
F = 128
NCLS = 16
NC = 2
NS = 16
NW = NC * NS
K = 128
FH = F // NC
E_PAD = 327680
PAD_E = E_PAD - E
EB = E_PAD // K
BPT = EB // NW
BPC = EB // NS
N_ACC = 10240
NPT = N_ACC // NS
NZC = 128
N_PAD = N_ACC
NPS = N_PAD // NS

_mesh = plsc.VectorSubcoreMesh(core_axis_name="c", subcore_axis_name="s")


@functools.partial(
    pl.kernel,
    out_type=(
        jax.ShapeDtypeStruct((EB, K), jnp.float32),
        jax.ShapeDtypeStruct((NC * N_PAD,), jnp.float32),
        jax.ShapeDtypeStruct((NC * N_PAD,), jnp.float32),
    ),
    mesh=_mesh,
    compiler_params=pltpu.CompilerParams(use_tc_tiling_on_sc=False),
    scratch_types=(
        pltpu.VMEM((BPT, K), jnp.float32),
        pltpu.VMEM((BPT, K), jnp.int32),
        pltpu.VMEM((BPT, K), jnp.float32),
        pltpu.VMEM((K,), jnp.float32),
        pltpu.VMEM((NPS,), jnp.float32),
        pltpu.VMEM_SHARED((N_PAD,), jnp.float32),
        pltpu.VMEM_SHARED((N_PAD,), jnp.float32),
    ),
)
def _deg_kernel(params2_h, dst2_h, ew2_h, deg1_h, deg3_h,
                pbuf, dbuf, ebuf, ones_v, zb, acc1, acc3):
    c = lax.axis_index("c")
    s = lax.axis_index("s")
    wid = s * NC + c

    def zbody(i, carry):
        zb[pl.ds(i * 16, 16)] = jnp.zeros((16,), jnp.float32)
        return carry

    lax.fori_loop(0, NPS // 16, zbody, 0)
    for i in range(K // 16):
        ones_v[pl.ds(i * 16, 16)] = jnp.full((16,), 1.0, jnp.float32)

    pltpu.sync_copy(zb, acc1.at[pl.ds(s * NPS, NPS)])
    pltpu.sync_copy(zb, acc3.at[pl.ds(s * NPS, NPS)])
    plsc.subcore_barrier()

    row0 = wid * BPT
    pltpu.sync_copy(params2_h.at[pl.ds(row0, BPT)], pbuf)
    pltpu.sync_copy(dst2_h.at[pl.ds(row0, BPT)], dbuf)

    def body(b, carry):
        for cc in range(K // 16):
            sl = pl.ds(cc * 16, 16)
            v = pbuf[b, sl]
            ebuf[b, sl] = 1.0 / (1.0 + jnp.exp(-v))
        pltpu.sync_copy(ebuf.at[b], acc1.at[dbuf.at[b]], add=True)
        pltpu.sync_copy(ones_v, acc3.at[dbuf.at[b]], add=True)
        return carry

    lax.fori_loop(0, BPT, body, 0)
    pltpu.sync_copy(ebuf, ew2_h.at[pl.ds(row0, BPT)])
    plsc.subcore_barrier()

    base = c * N_PAD + s * NPS
    pltpu.sync_copy(acc1.at[pl.ds(s * NPS, NPS)], zb)
    pltpu.sync_copy(zb, deg1_h.at[pl.ds(base, NPS)])
    pltpu.sync_copy(acc3.at[pl.ds(s * NPS, NPS)], zb)
    pltpu.sync_copy(zb, deg3_h.at[pl.ds(base, NPS)])


def _make_spmm(weighted):
    @functools.partial(
        pl.kernel,
        out_type=jax.ShapeDtypeStruct((NC, N_ACC, FH), jnp.float32),
        mesh=_mesh,
        compiler_params=pltpu.CompilerParams(use_tc_tiling_on_sc=False),
        scratch_types=(
            pltpu.VMEM((BPC, K), jnp.int32),
            pltpu.VMEM((BPC, K), jnp.int32),
            pltpu.VMEM((BPC, K), jnp.float32),
            pltpu.VMEM((K, FH), jnp.float32),
            pltpu.VMEM((K, FH), jnp.float32),
            pltpu.VMEM((NZC, FH), jnp.float32),
            pltpu.VMEM_SHARED((N_ACC, FH), jnp.float32),
            pltpu.SemaphoreType.DMA,
            pltpu.SemaphoreType.DMA,
        ),
    )
    def spmm(y_h, src2_h, dst2_h, w2_h, s_h,
             sbuf, dbuf, wbuf, rows0, rows1, zb, acc, sem0, sem1):
        c = lax.axis_index("c")
        s_idx = lax.axis_index("s")

        def zrow(i, carry):
            for cc in range(FH // 16):
                zb[i, pl.ds(cc * 16, 16)] = jnp.zeros((16,), jnp.float32)
            return carry

        lax.fori_loop(0, NZC, zrow, 0)
        r0 = s_idx * NPT
        for j in range(NPT // NZC):
            pltpu.sync_copy(zb, acc.at[pl.ds(r0 + j * NZC, NZC)])
        plsc.subcore_barrier()

        row0 = s_idx * BPC
        pltpu.sync_copy(src2_h.at[pl.ds(row0, BPC)], sbuf)
        pltpu.sync_copy(dst2_h.at[pl.ds(row0, BPC)], dbuf)
        if weighted:
            pltpu.sync_copy(w2_h.at[pl.ds(row0, BPC)], wbuf)

        def scale_rows(rows, b):
            def scale(g, inner):
                wv = wbuf[b, pl.ds(g * 16, 16)]
                for j in range(16):
                    wj = wv[j]
                    for cc in range(FH // 16):
                        sl = pl.ds(cc * 16, 16)
                        rows[g * 16 + j, sl] = rows[g * 16 + j, sl] * wj
                return inner

            lax.fori_loop(0, K // 16, scale, 0)

        def stage(b, rows, sem, rows_nxt, sem_nxt):
            pltpu.make_async_copy(y_h.at[c].at[sbuf.at[b]], rows, sem).wait()

            @pl.when(b + 1 < BPC)
            def _():
                pltpu.async_copy(y_h.at[c].at[sbuf.at[b + 1]], rows_nxt,
                                 sem_nxt)

            if weighted:
                scale_rows(rows, b)
            pltpu.sync_copy(rows, acc.at[dbuf.at[b]], add=True)

        pltpu.async_copy(y_h.at[c].at[sbuf.at[0]], rows0, sem0)

        def body(i2, carry):
            b0 = i2 * 2
            stage(b0, rows0, sem0, rows1, sem1)
            stage(b0 + 1, rows1, sem1, rows0, sem0)
            return carry

        lax.fori_loop(0, BPC // 2, body, 0)
        plsc.subcore_barrier()

        for j in range(NPT // NZC):
            sl = pl.ds(r0 + j * NZC, NZC)
            pltpu.sync_copy(acc.at[sl], zb)
            pltpu.sync_copy(zb, s_h.at[c, sl])

    return spmm


_spmm_w = _make_spmm(True)
_spmm_u = _make_spmm(False)


BR = 1000


def _tc_a_body(deg1_r, deg3_r, x_r, w1_r, d1_o, d3_o, y0_o):
    d1 = lax.rsqrt(deg1_r[:, 0:1] + deg1_r[:, 1:2] + 1.0)
    d3 = lax.rsqrt(deg3_r[:, 0:1] + deg3_r[:, 1:2] + 1.0)
    d1_o[...] = d1
    d3_o[...] = d3
    y0 = d1 * jnp.dot(x_r[...], w1_r[...], preferred_element_type=jnp.float32)
    y0_o[0] = y0[:, :FH]
    y0_o[1] = y0[:, FH:]


def _tc_a(deg1T, deg3T, x, W1):
    return pl.pallas_call(
        _tc_a_body,
        grid=(N // BR,),
        in_specs=[
            pl.BlockSpec((BR, NC), lambda i: (i, 0)),
            pl.BlockSpec((BR, NC), lambda i: (i, 0)),
            pl.BlockSpec((BR, F), lambda i: (i, 0)),
            pl.BlockSpec((F, F), lambda i: (0, 0)),
        ],
        out_specs=[
            pl.BlockSpec((BR, 1), lambda i: (i, 0)),
            pl.BlockSpec((BR, 1), lambda i: (i, 0)),
            pl.BlockSpec((NC, BR, FH), lambda i: (0, i, 0)),
        ],
        out_shape=[
            jax.ShapeDtypeStruct((N, 1), jnp.float32),
            jax.ShapeDtypeStruct((N, 1), jnp.float32),
            jax.ShapeDtypeStruct((NC, N, FH), jnp.float32),
        ],
    )(deg1T, deg3T, x, W1)


def _tc_b_body(s_r, y_r, dp_r, b_r, w_r, dn_r, x_o, yn_o):
    comb = jnp.concatenate([s_r[0] + y_r[0], s_r[1] + y_r[1]], axis=1)
    xl = jnp.maximum(dp_r[...] * comb + b_r[...], 0.0)
    x_o[...] = xl
    yn = dn_r[...] * jnp.dot(xl, w_r[...], preferred_element_type=jnp.float32)
    yn_o[0] = yn[:, :FH]
    yn_o[1] = yn[:, FH:]


def _tc_b(s, y_prev, d_prev, b_prev, W_next, d_next):
    return pl.pallas_call(
        _tc_b_body,
        grid=(N // BR,),
        in_specs=[
            pl.BlockSpec((NC, BR, FH), lambda i: (0, i, 0)),
            pl.BlockSpec((NC, BR, FH), lambda i: (0, i, 0)),
            pl.BlockSpec((BR, 1), lambda i: (i, 0)),
            pl.BlockSpec((1, F), lambda i: (0, 0)),
            pl.BlockSpec((F, F), lambda i: (0, 0)),
            pl.BlockSpec((BR, 1), lambda i: (i, 0)),
        ],
        out_specs=[
            pl.BlockSpec((BR, F), lambda i: (i, 0)),
            pl.BlockSpec((NC, BR, FH), lambda i: (0, i, 0)),
        ],
        out_shape=[
            jax.ShapeDtypeStruct((N, F), jnp.float32),
            jax.ShapeDtypeStruct((NC, N, FH), jnp.float32),
        ],
    )(s, y_prev, d_prev, b_prev, W_next, d_next)


def _tc_d_body(s_r, y_r, d3_r, b_r, x1_r, x2_r, wl_r, bl_r, out_o):
    x3 = d3_r[...] * jnp.concatenate(
        [s_r[0] + y_r[0], s_r[1] + y_r[1]], axis=1) + b_r[...]
    wl = wl_r[...]
    h = (jnp.dot(x1_r[...], wl[0:F], preferred_element_type=jnp.float32)
         + jnp.dot(x2_r[...], wl[F:2 * F], preferred_element_type=jnp.float32)
         + jnp.dot(x3, wl[2 * F:3 * F], preferred_element_type=jnp.float32)
         + bl_r[...])
    m = jnp.max(h, axis=1, keepdims=True)
    e = jnp.exp(h - m)
    lse = jnp.log(jnp.sum(e, axis=1, keepdims=True))
    out_o[...] = h - m - lse


def _tc_d(s3, y2, d3, b3, x1, x2, Wlin, blin):
    return pl.pallas_call(
        _tc_d_body,
        grid=(N // BR,),
        in_specs=[
            pl.BlockSpec((NC, BR, FH), lambda i: (0, i, 0)),
            pl.BlockSpec((NC, BR, FH), lambda i: (0, i, 0)),
            pl.BlockSpec((BR, 1), lambda i: (i, 0)),
            pl.BlockSpec((1, F), lambda i: (0, 0)),
            pl.BlockSpec((BR, F), lambda i: (i, 0)),
            pl.BlockSpec((BR, F), lambda i: (i, 0)),
            pl.BlockSpec((3 * F, NCLS), lambda i: (0, 0)),
            pl.BlockSpec((1, NCLS), lambda i: (0, 0)),
        ],
        out_specs=pl.BlockSpec((BR, NCLS), lambda i: (i, 0)),
        out_shape=jax.ShapeDtypeStruct((N, NCLS), jnp.float32),
    )(s3, y2, d3, b3, x1, x2, Wlin, blin)


def kernel(x, edge_index, edge_weight_params, W1, b1, W2, b2, W3, b3, Wlin, blin):
    pad_src = jnp.zeros((PAD_E,), jnp.int32)
    pad_dst = jnp.full((PAD_E,), N, jnp.int32)
    src2 = jnp.concatenate([edge_index[0], pad_src]).reshape(EB, K)
    dst2 = jnp.concatenate([edge_index[1], pad_dst]).reshape(EB, K)
    p2 = jnp.concatenate(
        [edge_weight_params, jnp.zeros((PAD_E,), jnp.float32)]).reshape(EB, K)

    ew2, deg1f, deg3f = _deg_kernel(p2, dst2)
    deg1T = deg1f.reshape(NC, N_PAD)[:, :N].T
    deg3T = deg3f.reshape(NC, N_PAD)[:, :N].T

    d1, d3, y0 = _tc_a(deg1T, deg3T, x, W1)

    s1 = _spmm_w(y0, src2, dst2, ew2)
    x1, y1 = _tc_b(s1, y0, d1, b1.reshape(1, F), W2, d1)

    s2 = _spmm_w(y1, src2, dst2, ew2)
    x2, y2 = _tc_b(s2, y1, d1, b2.reshape(1, F), W3, d3)

    s3 = _spmm_u(y2, src2, dst2, ew2)
    out = _tc_d(s3, y2, d3, b3.reshape(1, F), x1, x2,
                Wlin, blin.reshape(1, NCLS))
    return out


# speedup vs baseline: 11.1599x; 1.0690x over previous
import functools

import jax
import jax.numpy as jnp
from jax import lax
from jax.experimental import pallas as pl
from jax.experimental.pallas import tpu as pltpu
from jax.experimental.pallas import tpu_sc as plsc

N = 10000
E = 320000
F = 128
NCLS = 16
NC = 2
NS = 16
NW = NC * NS
K = 128
FH = F // NC
E_PAD = 327680
PAD_E = E_PAD - E
EB = E_PAD // K
BPT = EB // NW
BPC = EB // NS
N_ACC = 10240
NPT = N_ACC // NS
NZC = 32
N_PAD = N_ACC
NPS = N_PAD // NS

_mesh = plsc.VectorSubcoreMesh(core_axis_name="c", subcore_axis_name="s")


@functools.partial(
    pl.kernel,
    out_type=(
        jax.ShapeDtypeStruct((EB, K), jnp.float32),
        jax.ShapeDtypeStruct((NC * N_PAD,), jnp.float32),
        jax.ShapeDtypeStruct((NC * N_PAD,), jnp.float32),
    ),
    mesh=_mesh,
    compiler_params=pltpu.CompilerParams(use_tc_tiling_on_sc=False),
    scratch_types=(
        pltpu.VMEM((BPT, K), jnp.float32),
        pltpu.VMEM((BPT, K), jnp.int32),
        pltpu.VMEM((BPT, K), jnp.float32),
        pltpu.VMEM((K,), jnp.float32),
        pltpu.VMEM((NPS,), jnp.float32),
        pltpu.VMEM_SHARED((N_PAD,), jnp.float32),
        pltpu.VMEM_SHARED((N_PAD,), jnp.float32),
    ),
)
def _deg_kernel(params2_h, dst2_h, ew2_h, deg1_h, deg3_h,
                pbuf, dbuf, ebuf, ones_v, zb, acc1, acc3):
    c = lax.axis_index("c")
    s = lax.axis_index("s")
    wid = s * NC + c

    def zbody(i, carry):
        zb[pl.ds(i * 16, 16)] = jnp.zeros((16,), jnp.float32)
        return carry

    lax.fori_loop(0, NPS // 16, zbody, 0)
    for i in range(K // 16):
        ones_v[pl.ds(i * 16, 16)] = jnp.full((16,), 1.0, jnp.float32)

    pltpu.sync_copy(zb, acc1.at[pl.ds(s * NPS, NPS)])
    pltpu.sync_copy(zb, acc3.at[pl.ds(s * NPS, NPS)])
    plsc.subcore_barrier()

    row0 = wid * BPT
    pltpu.sync_copy(params2_h.at[pl.ds(row0, BPT)], pbuf)
    pltpu.sync_copy(dst2_h.at[pl.ds(row0, BPT)], dbuf)

    def body(b, carry):
        for cc in range(K // 16):
            sl = pl.ds(cc * 16, 16)
            v = pbuf[b, sl]
            ebuf[b, sl] = 1.0 / (1.0 + jnp.exp(-v))
        pltpu.sync_copy(ebuf.at[b], acc1.at[dbuf.at[b]], add=True)
        pltpu.sync_copy(ones_v, acc3.at[dbuf.at[b]], add=True)
        return carry

    lax.fori_loop(0, BPT, body, 0)
    pltpu.sync_copy(ebuf, ew2_h.at[pl.ds(row0, BPT)])
    plsc.subcore_barrier()

    base = c * N_PAD + s * NPS
    pltpu.sync_copy(acc1.at[pl.ds(s * NPS, NPS)], zb)
    pltpu.sync_copy(zb, deg1_h.at[pl.ds(base, NPS)])
    pltpu.sync_copy(acc3.at[pl.ds(s * NPS, NPS)], zb)
    pltpu.sync_copy(zb, deg3_h.at[pl.ds(base, NPS)])


def _make_spmm(weighted):
    @functools.partial(
        pl.kernel,
        out_type=jax.ShapeDtypeStruct((NC, N_ACC, FH), jnp.float32),
        mesh=_mesh,
        compiler_params=pltpu.CompilerParams(use_tc_tiling_on_sc=False),
        scratch_types=(
            pltpu.VMEM((BPC, K), jnp.int32),
            pltpu.VMEM((BPC, K), jnp.int32),
            pltpu.VMEM((BPC, K), jnp.float32),
            pltpu.VMEM((K, FH), jnp.float32),
            pltpu.VMEM((K, FH), jnp.float32),
            pltpu.VMEM((K, FH), jnp.float32),
            pltpu.VMEM((NZC, FH), jnp.float32),
            pltpu.VMEM_SHARED((N_ACC, FH), jnp.float32),
            pltpu.SemaphoreType.DMA,
            pltpu.SemaphoreType.DMA,
            pltpu.SemaphoreType.DMA,
            pltpu.SemaphoreType.DMA,
            pltpu.SemaphoreType.DMA,
            pltpu.SemaphoreType.DMA,
        ),
    )
    def spmm(y_h, src2_h, dst2_h, w2_h, s_h,
             sbuf, dbuf, wbuf, r0b, r1b, r2b, zb, acc,
             g0, g1, g2, t0, t1, t2):
        c = lax.axis_index("c")
        s_idx = lax.axis_index("s")
        rbufs = (r0b, r1b, r2b)
        gsems = (g0, g1, g2)
        tsems = (t0, t1, t2)

        def zrow(i, carry):
            for cc in range(FH // 16):
                zb[i, pl.ds(cc * 16, 16)] = jnp.zeros((16,), jnp.float32)
            return carry

        lax.fori_loop(0, NZC, zrow, 0)
        r0 = s_idx * NPT
        for j in range(NPT // NZC):
            pltpu.sync_copy(zb, acc.at[pl.ds(r0 + j * NZC, NZC)])
        plsc.subcore_barrier()

        row0 = s_idx * BPC
        pltpu.sync_copy(src2_h.at[pl.ds(row0, BPC)], sbuf)
        pltpu.sync_copy(dst2_h.at[pl.ds(row0, BPC)], dbuf)
        if weighted:
            pltpu.sync_copy(w2_h.at[pl.ds(row0, BPC)], wbuf)

        def scale_rows(rows, b):
            def scale(g, inner):
                wv = wbuf[b, pl.ds(g * 16, 16)]
                for j in range(16):
                    wj = wv[j]
                    for cc in range(FH // 16):
                        sl = pl.ds(cc * 16, 16)
                        rows[g * 16 + j, sl] = rows[g * 16 + j, sl] * wj
                return inner

            lax.fori_loop(0, K // 16, scale, 0)

        pltpu.async_copy(y_h.at[c].at[sbuf.at[0]], r0b, g0)
        pltpu.async_copy(y_h.at[c].at[sbuf.at[1]], r1b, g1)

        def stage(i3, sidx):
            b = i3 * 3 + sidx
            p = sidx
            q = (sidx + 2) % 3
            rows = rbufs[p]
            pltpu.make_async_copy(y_h.at[c].at[sbuf.at[b]], rows,
                                  gsems[p]).wait()
            if sidx == 0:
                @pl.when(b >= 1)
                def _():
                    pltpu.make_async_copy(
                        rbufs[q], acc.at[dbuf.at[b - 1]], tsems[q]).wait()
            else:
                pltpu.make_async_copy(
                    rbufs[q], acc.at[dbuf.at[b - 1]], tsems[q]).wait()

            @pl.when(b + 2 < BPC)
            def _():
                pltpu.async_copy(y_h.at[c].at[sbuf.at[b + 2]], rbufs[q],
                                 gsems[q])

            if weighted:
                scale_rows(rows, b)
            pltpu.async_copy(rows, acc.at[dbuf.at[b]], tsems[p], add=True)

        def body(i3, carry):
            for sidx in range(3):
                stage(i3, sidx)
            return carry

        lax.fori_loop(0, BPC // 3, body, 0)
        stage(BPC // 3, 0)
        pltpu.make_async_copy(
            rbufs[(BPC - 1) % 3], acc.at[dbuf.at[BPC - 1]],
            tsems[(BPC - 1) % 3]).wait()
        plsc.subcore_barrier()

        for j in range(NPT // NZC):
            sl = pl.ds(r0 + j * NZC, NZC)
            pltpu.sync_copy(acc.at[sl], zb)
            pltpu.sync_copy(zb, s_h.at[c, sl])

    return spmm


_spmm_w = _make_spmm(True)
_spmm_u = _make_spmm(False)


BR = 1000


def _tc_a_body(deg1_r, deg3_r, x_r, w1_r, d1_o, d3_o, y0_o):
    d1 = lax.rsqrt(deg1_r[:, 0:1] + deg1_r[:, 1:2] + 1.0)
    d3 = lax.rsqrt(deg3_r[:, 0:1] + deg3_r[:, 1:2] + 1.0)
    d1_o[...] = d1
    d3_o[...] = d3
    y0 = d1 * jnp.dot(x_r[...], w1_r[...], preferred_element_type=jnp.float32)
    y0_o[0] = y0[:, :FH]
    y0_o[1] = y0[:, FH:]


def _tc_a(deg1T, deg3T, x, W1):
    return pl.pallas_call(
        _tc_a_body,
        grid=(N // BR,),
        in_specs=[
            pl.BlockSpec((BR, NC), lambda i: (i, 0)),
            pl.BlockSpec((BR, NC), lambda i: (i, 0)),
            pl.BlockSpec((BR, F), lambda i: (i, 0)),
            pl.BlockSpec((F, F), lambda i: (0, 0)),
        ],
        out_specs=[
            pl.BlockSpec((BR, 1), lambda i: (i, 0)),
            pl.BlockSpec((BR, 1), lambda i: (i, 0)),
            pl.BlockSpec((NC, BR, FH), lambda i: (0, i, 0)),
        ],
        out_shape=[
            jax.ShapeDtypeStruct((N, 1), jnp.float32),
            jax.ShapeDtypeStruct((N, 1), jnp.float32),
            jax.ShapeDtypeStruct((NC, N, FH), jnp.float32),
        ],
    )(deg1T, deg3T, x, W1)


def _tc_b_body(s_r, y_r, dp_r, b_r, w_r, dn_r, x_o, yn_o):
    comb = jnp.concatenate([s_r[0] + y_r[0], s_r[1] + y_r[1]], axis=1)
    xl = jnp.maximum(dp_r[...] * comb + b_r[...], 0.0)
    x_o[...] = xl
    yn = dn_r[...] * jnp.dot(xl, w_r[...], preferred_element_type=jnp.float32)
    yn_o[0] = yn[:, :FH]
    yn_o[1] = yn[:, FH:]


def _tc_b(s, y_prev, d_prev, b_prev, W_next, d_next):
    return pl.pallas_call(
        _tc_b_body,
        grid=(N // BR,),
        in_specs=[
            pl.BlockSpec((NC, BR, FH), lambda i: (0, i, 0)),
            pl.BlockSpec((NC, BR, FH), lambda i: (0, i, 0)),
            pl.BlockSpec((BR, 1), lambda i: (i, 0)),
            pl.BlockSpec((1, F), lambda i: (0, 0)),
            pl.BlockSpec((F, F), lambda i: (0, 0)),
            pl.BlockSpec((BR, 1), lambda i: (i, 0)),
        ],
        out_specs=[
            pl.BlockSpec((BR, F), lambda i: (i, 0)),
            pl.BlockSpec((NC, BR, FH), lambda i: (0, i, 0)),
        ],
        out_shape=[
            jax.ShapeDtypeStruct((N, F), jnp.float32),
            jax.ShapeDtypeStruct((NC, N, FH), jnp.float32),
        ],
    )(s, y_prev, d_prev, b_prev, W_next, d_next)


def _tc_d_body(s_r, y_r, d3_r, b_r, x1_r, x2_r, wl_r, bl_r, out_o):
    x3 = d3_r[...] * jnp.concatenate(
        [s_r[0] + y_r[0], s_r[1] + y_r[1]], axis=1) + b_r[...]
    wl = wl_r[...]
    h = (jnp.dot(x1_r[...], wl[0:F], preferred_element_type=jnp.float32)
         + jnp.dot(x2_r[...], wl[F:2 * F], preferred_element_type=jnp.float32)
         + jnp.dot(x3, wl[2 * F:3 * F], preferred_element_type=jnp.float32)
         + bl_r[...])
    m = jnp.max(h, axis=1, keepdims=True)
    e = jnp.exp(h - m)
    lse = jnp.log(jnp.sum(e, axis=1, keepdims=True))
    out_o[...] = h - m - lse


def _tc_d(s3, y2, d3, b3, x1, x2, Wlin, blin):
    return pl.pallas_call(
        _tc_d_body,
        grid=(N // BR,),
        in_specs=[
            pl.BlockSpec((NC, BR, FH), lambda i: (0, i, 0)),
            pl.BlockSpec((NC, BR, FH), lambda i: (0, i, 0)),
            pl.BlockSpec((BR, 1), lambda i: (i, 0)),
            pl.BlockSpec((1, F), lambda i: (0, 0)),
            pl.BlockSpec((BR, F), lambda i: (i, 0)),
            pl.BlockSpec((BR, F), lambda i: (i, 0)),
            pl.BlockSpec((3 * F, NCLS), lambda i: (0, 0)),
            pl.BlockSpec((1, NCLS), lambda i: (0, 0)),
        ],
        out_specs=pl.BlockSpec((BR, NCLS), lambda i: (i, 0)),
        out_shape=jax.ShapeDtypeStruct((N, NCLS), jnp.float32),
    )(s3, y2, d3, b3, x1, x2, Wlin, blin)


def kernel(x, edge_index, edge_weight_params, W1, b1, W2, b2, W3, b3, Wlin, blin):
    pad_src = jnp.zeros((PAD_E,), jnp.int32)
    pad_dst = jnp.full((PAD_E,), N, jnp.int32)
    src2 = jnp.concatenate([edge_index[0], pad_src]).reshape(EB, K)
    dst2 = jnp.concatenate([edge_index[1], pad_dst]).reshape(EB, K)
    p2 = jnp.concatenate(
        [edge_weight_params, jnp.zeros((PAD_E,), jnp.float32)]).reshape(EB, K)

    ew2, deg1f, deg3f = _deg_kernel(p2, dst2)
    deg1T = deg1f.reshape(NC, N_PAD)[:, :N].T
    deg3T = deg3f.reshape(NC, N_PAD)[:, :N].T

    d1, d3, y0 = _tc_a(deg1T, deg3T, x, W1)

    s1 = _spmm_w(y0, src2, dst2, ew2)
    x1, y1 = _tc_b(s1, y0, d1, b1.reshape(1, F), W2, d1)

    s2 = _spmm_w(y1, src2, dst2, ew2)
    x2, y2 = _tc_b(s2, y1, d1, b2.reshape(1, F), W3, d3)

    s3 = _spmm_u(y2, src2, dst2, ew2)
    out = _tc_d(s3, y2, d3, b3.reshape(1, F), x1, x2,
                Wlin, blin.reshape(1, NCLS))
    return out


# speedup vs baseline: 11.6886x; 1.0474x over previous
import functools

import jax
import jax.numpy as jnp
from jax import lax
from jax.experimental import pallas as pl
from jax.experimental.pallas import tpu as pltpu
from jax.experimental.pallas import tpu_sc as plsc

N = 10000
E = 320000
F = 128
NCLS = 16
NC = 2
NS = 16
NW = NC * NS
K = 128
FH = F // NC
E_PAD = 327680
PAD_E = E_PAD - E
EB = E_PAD // K
BPT = EB // NW
BPC = EB // NS
N_ACC = 10240
NPT = N_ACC // NS
NZC = 32
N_PAD = N_ACC
NPS = N_PAD // NS

_mesh = plsc.VectorSubcoreMesh(core_axis_name="c", subcore_axis_name="s")


@functools.partial(
    pl.kernel,
    out_type=(
        jax.ShapeDtypeStruct((EB, K), jnp.float32),
        jax.ShapeDtypeStruct((NC * N_PAD,), jnp.float32),
        jax.ShapeDtypeStruct((NC * N_PAD,), jnp.float32),
    ),
    mesh=_mesh,
    compiler_params=pltpu.CompilerParams(use_tc_tiling_on_sc=False),
    scratch_types=(
        pltpu.VMEM((BPT, K), jnp.float32),
        pltpu.VMEM((BPT, K), jnp.int32),
        pltpu.VMEM((BPT, K), jnp.float32),
        pltpu.VMEM((K,), jnp.float32),
        pltpu.VMEM((NPS,), jnp.float32),
        pltpu.VMEM_SHARED((N_PAD,), jnp.float32),
        pltpu.VMEM_SHARED((N_PAD,), jnp.float32),
    ),
)
def _deg_kernel(params2_h, dst2_h, ew2_h, deg1_h, deg3_h,
                pbuf, dbuf, ebuf, ones_v, zb, acc1, acc3):
    c = lax.axis_index("c")
    s = lax.axis_index("s")
    wid = s * NC + c

    def zbody(i, carry):
        zb[pl.ds(i * 16, 16)] = jnp.zeros((16,), jnp.float32)
        return carry

    lax.fori_loop(0, NPS // 16, zbody, 0)
    for i in range(K // 16):
        ones_v[pl.ds(i * 16, 16)] = jnp.full((16,), 1.0, jnp.float32)

    pltpu.sync_copy(zb, acc1.at[pl.ds(s * NPS, NPS)])
    pltpu.sync_copy(zb, acc3.at[pl.ds(s * NPS, NPS)])
    plsc.subcore_barrier()

    row0 = wid * BPT
    pltpu.sync_copy(params2_h.at[pl.ds(row0, BPT)], pbuf)
    pltpu.sync_copy(dst2_h.at[pl.ds(row0, BPT)], dbuf)

    def body(b, carry):
        for cc in range(K // 16):
            sl = pl.ds(cc * 16, 16)
            v = pbuf[b, sl]
            ebuf[b, sl] = 1.0 / (1.0 + jnp.exp(-v))
        pltpu.sync_copy(ebuf.at[b], acc1.at[dbuf.at[b]], add=True)
        pltpu.sync_copy(ones_v, acc3.at[dbuf.at[b]], add=True)
        return carry

    lax.fori_loop(0, BPT, body, 0)
    pltpu.sync_copy(ebuf, ew2_h.at[pl.ds(row0, BPT)])
    plsc.subcore_barrier()

    base = c * N_PAD + s * NPS
    pltpu.sync_copy(acc1.at[pl.ds(s * NPS, NPS)], zb)
    pltpu.sync_copy(zb, deg1_h.at[pl.ds(base, NPS)])
    pltpu.sync_copy(acc3.at[pl.ds(s * NPS, NPS)], zb)
    pltpu.sync_copy(zb, deg3_h.at[pl.ds(base, NPS)])


def _make_spmm(weighted):
    @functools.partial(
        pl.kernel,
        out_type=jax.ShapeDtypeStruct((NC, N_ACC, FH), jnp.float32),
        mesh=_mesh,
        compiler_params=pltpu.CompilerParams(use_tc_tiling_on_sc=False),
        scratch_types=(
            pltpu.VMEM((BPC, K), jnp.int32),
            pltpu.VMEM((BPC, K), jnp.int32),
            pltpu.VMEM((BPC, K), jnp.float32),
            pltpu.VMEM((K, FH), jnp.float32),
            pltpu.VMEM((K, FH), jnp.float32),
            pltpu.VMEM((K, FH), jnp.float32),
            pltpu.VMEM((NZC, FH), jnp.float32),
            pltpu.VMEM_SHARED((N_ACC, FH), jnp.float32),
            pltpu.SemaphoreType.DMA,
            pltpu.SemaphoreType.DMA,
            pltpu.SemaphoreType.DMA,
            pltpu.SemaphoreType.DMA,
            pltpu.SemaphoreType.DMA,
            pltpu.SemaphoreType.DMA,
        ),
    )
    def spmm(y_h, src2_h, dst2_h, w2_h, s_h,
             sbuf, dbuf, wbuf, r0b, r1b, r2b, zb, acc,
             g0, g1, g2, t0, t1, t2):
        c = lax.axis_index("c")
        s_idx = lax.axis_index("s")
        rbufs = (r0b, r1b, r2b)
        gsems = (g0, g1, g2)
        tsems = (t0, t1, t2)

        def zrow(i, carry):
            for cc in range(FH // 16):
                zb[i, pl.ds(cc * 16, 16)] = jnp.zeros((16,), jnp.float32)
            return carry

        lax.fori_loop(0, NZC, zrow, 0)
        r0 = s_idx * NPT
        for j in range(NPT // NZC):
            pltpu.sync_copy(zb, acc.at[pl.ds(r0 + j * NZC, NZC)])
        plsc.subcore_barrier()

        row0 = s_idx * BPC
        pltpu.sync_copy(src2_h.at[pl.ds(row0, BPC)], sbuf)
        pltpu.sync_copy(dst2_h.at[pl.ds(row0, BPC)], dbuf)
        if weighted:
            pltpu.sync_copy(w2_h.at[pl.ds(row0, BPC)], wbuf)

        def scale_rows(rows, b):
            def scale(g, inner):
                wv = wbuf[b, pl.ds(g * 16, 16)]
                for j in range(16):
                    wj = wv[j]
                    for cc in range(FH // 16):
                        sl = pl.ds(cc * 16, 16)
                        rows[g * 16 + j, sl] = rows[g * 16 + j, sl] * wj
                return inner

            lax.fori_loop(0, K // 16, scale, 0)

        pltpu.async_copy(y_h.at[c].at[sbuf.at[0]], r0b, g0)
        pltpu.async_copy(y_h.at[c].at[sbuf.at[1]], r1b, g1)

        def stage(i3, sidx):
            b = i3 * 3 + sidx
            p = sidx
            q = (sidx + 2) % 3
            rows = rbufs[p]
            pltpu.make_async_copy(y_h.at[c].at[sbuf.at[b]], rows,
                                  gsems[p]).wait()
            if weighted:
                scale_rows(rows, b)
            if sidx == 0:
                @pl.when(b >= 1)
                def _():
                    pltpu.make_async_copy(
                        rbufs[q], acc.at[dbuf.at[b - 1]], tsems[q]).wait()
            else:
                pltpu.make_async_copy(
                    rbufs[q], acc.at[dbuf.at[b - 1]], tsems[q]).wait()

            @pl.when(b + 2 < BPC)
            def _():
                pltpu.async_copy(y_h.at[c].at[sbuf.at[b + 2]], rbufs[q],
                                 gsems[q])

            pltpu.async_copy(rows, acc.at[dbuf.at[b]], tsems[p], add=True)

        def body(i3, carry):
            for sidx in range(3):
                stage(i3, sidx)
            return carry

        lax.fori_loop(0, BPC // 3, body, 0)
        stage(BPC // 3, 0)
        pltpu.make_async_copy(
            rbufs[(BPC - 1) % 3], acc.at[dbuf.at[BPC - 1]],
            tsems[(BPC - 1) % 3]).wait()
        plsc.subcore_barrier()

        for j in range(NPT // NZC):
            sl = pl.ds(r0 + j * NZC, NZC)
            pltpu.sync_copy(acc.at[sl], zb)
            pltpu.sync_copy(zb, s_h.at[c, sl])

    return spmm


_spmm_w = _make_spmm(True)
_spmm_u = _make_spmm(False)


BR = 1000


def _tc_a_body(deg1_r, deg3_r, x_r, w1_r, d1_o, d3_o, y0_o):
    d1 = lax.rsqrt(deg1_r[:, 0:1] + deg1_r[:, 1:2] + 1.0)
    d3 = lax.rsqrt(deg3_r[:, 0:1] + deg3_r[:, 1:2] + 1.0)
    d1_o[...] = d1
    d3_o[...] = d3
    y0 = d1 * jnp.dot(x_r[...], w1_r[...], preferred_element_type=jnp.float32)
    y0_o[0] = y0[:, :FH]
    y0_o[1] = y0[:, FH:]


def _tc_a(deg1T, deg3T, x, W1):
    return pl.pallas_call(
        _tc_a_body,
        grid=(N // BR,),
        in_specs=[
            pl.BlockSpec((BR, NC), lambda i: (i, 0)),
            pl.BlockSpec((BR, NC), lambda i: (i, 0)),
            pl.BlockSpec((BR, F), lambda i: (i, 0)),
            pl.BlockSpec((F, F), lambda i: (0, 0)),
        ],
        out_specs=[
            pl.BlockSpec((BR, 1), lambda i: (i, 0)),
            pl.BlockSpec((BR, 1), lambda i: (i, 0)),
            pl.BlockSpec((NC, BR, FH), lambda i: (0, i, 0)),
        ],
        out_shape=[
            jax.ShapeDtypeStruct((N, 1), jnp.float32),
            jax.ShapeDtypeStruct((N, 1), jnp.float32),
            jax.ShapeDtypeStruct((NC, N, FH), jnp.float32),
        ],
    )(deg1T, deg3T, x, W1)


def _tc_b_body(s_r, y_r, dp_r, b_r, w_r, dn_r, x_o, yn_o):
    comb = jnp.concatenate([s_r[0] + y_r[0], s_r[1] + y_r[1]], axis=1)
    xl = jnp.maximum(dp_r[...] * comb + b_r[...], 0.0)
    x_o[...] = xl
    yn = dn_r[...] * jnp.dot(xl, w_r[...], preferred_element_type=jnp.float32)
    yn_o[0] = yn[:, :FH]
    yn_o[1] = yn[:, FH:]


def _tc_b(s, y_prev, d_prev, b_prev, W_next, d_next):
    return pl.pallas_call(
        _tc_b_body,
        grid=(N // BR,),
        in_specs=[
            pl.BlockSpec((NC, BR, FH), lambda i: (0, i, 0)),
            pl.BlockSpec((NC, BR, FH), lambda i: (0, i, 0)),
            pl.BlockSpec((BR, 1), lambda i: (i, 0)),
            pl.BlockSpec((1, F), lambda i: (0, 0)),
            pl.BlockSpec((F, F), lambda i: (0, 0)),
            pl.BlockSpec((BR, 1), lambda i: (i, 0)),
        ],
        out_specs=[
            pl.BlockSpec((BR, F), lambda i: (i, 0)),
            pl.BlockSpec((NC, BR, FH), lambda i: (0, i, 0)),
        ],
        out_shape=[
            jax.ShapeDtypeStruct((N, F), jnp.float32),
            jax.ShapeDtypeStruct((NC, N, FH), jnp.float32),
        ],
    )(s, y_prev, d_prev, b_prev, W_next, d_next)


def _tc_d_body(s_r, y_r, d3_r, b_r, x1_r, x2_r, wl_r, bl_r, out_o):
    x3 = d3_r[...] * jnp.concatenate(
        [s_r[0] + y_r[0], s_r[1] + y_r[1]], axis=1) + b_r[...]
    wl = wl_r[...]
    h = (jnp.dot(x1_r[...], wl[0:F], preferred_element_type=jnp.float32)
         + jnp.dot(x2_r[...], wl[F:2 * F], preferred_element_type=jnp.float32)
         + jnp.dot(x3, wl[2 * F:3 * F], preferred_element_type=jnp.float32)
         + bl_r[...])
    m = jnp.max(h, axis=1, keepdims=True)
    e = jnp.exp(h - m)
    lse = jnp.log(jnp.sum(e, axis=1, keepdims=True))
    out_o[...] = h - m - lse


def _tc_d(s3, y2, d3, b3, x1, x2, Wlin, blin):
    return pl.pallas_call(
        _tc_d_body,
        grid=(N // BR,),
        in_specs=[
            pl.BlockSpec((NC, BR, FH), lambda i: (0, i, 0)),
            pl.BlockSpec((NC, BR, FH), lambda i: (0, i, 0)),
            pl.BlockSpec((BR, 1), lambda i: (i, 0)),
            pl.BlockSpec((1, F), lambda i: (0, 0)),
            pl.BlockSpec((BR, F), lambda i: (i, 0)),
            pl.BlockSpec((BR, F), lambda i: (i, 0)),
            pl.BlockSpec((3 * F, NCLS), lambda i: (0, 0)),
            pl.BlockSpec((1, NCLS), lambda i: (0, 0)),
        ],
        out_specs=pl.BlockSpec((BR, NCLS), lambda i: (i, 0)),
        out_shape=jax.ShapeDtypeStruct((N, NCLS), jnp.float32),
    )(s3, y2, d3, b3, x1, x2, Wlin, blin)


def kernel(x, edge_index, edge_weight_params, W1, b1, W2, b2, W3, b3, Wlin, blin):
    pad_src = jnp.zeros((PAD_E,), jnp.int32)
    pad_dst = jnp.full((PAD_E,), N, jnp.int32)
    src2 = jnp.concatenate([edge_index[0], pad_src]).reshape(EB, K)
    dst2 = jnp.concatenate([edge_index[1], pad_dst]).reshape(EB, K)
    p2 = jnp.concatenate(
        [edge_weight_params, jnp.zeros((PAD_E,), jnp.float32)]).reshape(EB, K)

    ew2, deg1f, deg3f = _deg_kernel(p2, dst2)
    deg1T = deg1f.reshape(NC, N_PAD)[:, :N].T
    deg3T = deg3f.reshape(NC, N_PAD)[:, :N].T

    d1, d3, y0 = _tc_a(deg1T, deg3T, x, W1)

    s1 = _spmm_w(y0, src2, dst2, ew2)
    x1, y1 = _tc_b(s1, y0, d1, b1.reshape(1, F), W2, d1)

    s2 = _spmm_w(y1, src2, dst2, ew2)
    x2, y2 = _tc_b(s2, y1, d1, b2.reshape(1, F), W3, d3)

    s3 = _spmm_u(y2, src2, dst2, ew2)
    out = _tc_d(s3, y2, d3, b3.reshape(1, F), x1, x2,
                Wlin, blin.reshape(1, NCLS))
    return out
